# baseline (speedup 1.0000x reference)
import functools

import jax
import jax.numpy as jnp
from jax.experimental import pallas as pl
from jax.experimental.pallas import tpu as pltpu
from jax.experimental.pallas import tpu_sc as plsc

_AEV = 1008
_AEVP = 1024
_N = 100000
_BM = 512
_NB = 204
_C = _BM * _NB
_ROWS = 8
_COLS = 12512
_NPAD = _ROWS * _COLS
_H1, _H2, _H3 = 256, 192, 160
_SW = 40
_NSUB = 32
_GW = _C // _NSUB


def _celu(x):
    return jnp.where(x > 0, x, 0.1 * (jnp.exp(x * 10.0) - 1.0))


def _cumsum(x, axis):
    n = x.shape[axis]
    k = 1
    while k < n:
        zshape = list(x.shape)
        zshape[axis] = k
        pad = jnp.zeros(zshape, x.dtype)
        if axis == 1:
            x = x + jnp.concatenate([pad, x[:, :n - k]], axis=1)
        else:
            x = x + jnp.concatenate([pad, x[:n - k, :]], axis=0)
        k *= 2
    return x


_PR = 512


_HW = _AEVP // 2


def _pad_half(aev_t, half):
    def body(xt_ref, o_ref):
        o_ref[...] = xt_ref[...].T
        if half == 1:
            o_ref[:, _AEV - _HW:] = jnp.zeros(
                (_PR, _AEVP - _AEV), jnp.float32)

    nsteps = (_N + _PR - 1) // _PR
    return pl.pallas_call(
        body,
        grid=(nsteps,),
        in_specs=[pl.BlockSpec((_HW, _PR), lambda i, h=half: (h, i))],
        out_specs=pl.BlockSpec((_PR, _HW), lambda i: (i, 0)),
        out_shape=jax.ShapeDtypeStruct((_N, _HW), jnp.float32),
    )(aev_t)


def _route_body(s_ref, pos_ref, be_ref):
    s = s_ref[...]
    pos = jnp.zeros(s.shape, jnp.int32)
    off = jnp.int32(0)
    offs = []
    for b in range(8):
        ohb = s == b
        oh = ohb.astype(jnp.float32)
        inc = _cumsum(oh, axis=1)
        row_tot = inc[:, _COLS - 1:_COLS]
        ro = _cumsum(row_tot, axis=0) - row_tot
        rank_excl = inc + ro - oh
        cnt = (ro[7, 0] + row_tot[7, 0]).astype(jnp.int32)
        pos = jnp.where(ohb, off + rank_excl.astype(jnp.int32), pos)
        offs.append(off)
        off = off + ((cnt + _BM - 1) // _BM) * _BM
    pos_ref[...] = pos
    kidx = jax.lax.broadcasted_iota(jnp.int32, (1, 512), 1) * _BM
    be = jnp.zeros((1, 512), jnp.int32)
    for b in range(1, 8):
        be = be + (kidx >= offs[b]).astype(jnp.int32)
    be_ref[...] = be


def _mlp_body(be_ref, b4_ref, x0_ref, x1_ref, w1a_ref, w1b_ref, b1_ref,
              w2_ref, b2_ref, w3_ref, b3_ref, w4_ref, o_ref):
    cdims = (((1,), (1,)), ((), ()))
    x0 = x0_ref[...].astype(jnp.bfloat16)
    x1 = x1_ref[...].astype(jnp.bfloat16)
    h = jax.lax.dot_general(x0, w1a_ref[0], cdims,
                            preferred_element_type=jnp.float32)
    h = h + jax.lax.dot_general(x1, w1b_ref[0], cdims,
                                preferred_element_type=jnp.float32)
    h = _celu(h + b1_ref[0]).astype(jnp.bfloat16)
    h = jax.lax.dot_general(h, w2_ref[0], cdims,
                            preferred_element_type=jnp.float32)
    h = _celu(h + b2_ref[0]).astype(jnp.bfloat16)
    h = jax.lax.dot_general(h, w3_ref[0], cdims,
                            preferred_element_type=jnp.float32)
    h = _celu(h + b3_ref[0]).astype(jnp.bfloat16)
    y = jax.lax.dot_general(w4_ref[0], h, cdims,
                            preferred_element_type=jnp.float32)
    e = be_ref[pl.program_id(0)]
    o_ref[...] = (y + b4_ref[e]).reshape(1, 1, _BM)


def _sc_scatter_rows(aev, pos_row):
    mesh = plsc.VectorSubcoreMesh(core_axis_name="c", subcore_axis_name="s")

    @functools.partial(
        pl.kernel,
        out_type=jax.ShapeDtypeStruct((_C, _HW), jnp.float32),
        mesh=mesh,
        compiler_params=pltpu.CompilerParams(use_tc_tiling_on_sc=True))
    def scat(x_hbm, i_hbm, o_hbm):
        def body(x_vmem, i_vmem):
            pltpu.sync_copy(x_vmem, o_hbm.at[i_vmem.at[0]])

        pltpu.emit_pipeline(
            body,
            grid=(_N // _SW,),
            in_specs=[
                pl.BlockSpec((_SW, _HW), lambda i: (i, 0)),
                pl.BlockSpec((1, _SW), lambda i: (i, 0)),
            ],
            out_specs=[],
            core_axis_name=("c", "s"),
            dimension_semantics=(pltpu.PARALLEL,),
        )(x_hbm, i_hbm)

    return scat(aev, pos_row)


def _sc_gather_out(y_flat, idx_pad):
    mesh = plsc.VectorSubcoreMesh(core_axis_name="c", subcore_axis_name="s")

    @functools.partial(
        pl.kernel,
        out_type=jax.ShapeDtypeStruct((_C,), jnp.float32),
        mesh=mesh,
        compiler_params=pltpu.CompilerParams(needs_layout_passes=False),
        scratch_types=[
            pltpu.VMEM((_C,), jnp.float32),
            pltpu.VMEM((_GW,), jnp.int32),
            pltpu.VMEM((_GW,), jnp.float32),
            pltpu.SemaphoreType.DMA,
        ])
    def gat(y_hbm, i_hbm, o_hbm, y_vmem, idx_vmem, out_vmem, sem):
        c = jax.lax.axis_index("c")
        s = jax.lax.axis_index("s")
        u = c * 16 + s
        pltpu.async_copy(y_hbm, y_vmem, sem).wait()
        pltpu.async_copy(i_hbm.at[pl.ds(u * _GW, _GW)], idx_vmem, sem).wait()

        @pl.loop(0, _GW, step=16)
        def _(j):
            idx = idx_vmem[pl.ds(j, 16)]
            out_vmem[pl.ds(j, 16)] = plsc.load_gather(y_vmem, [idx])

        pltpu.async_copy(out_vmem, o_hbm.at[pl.ds(u * _GW, _GW)], sem).wait()

    return gat(y_flat, idx_pad)


def _stack_params(params):
    dims = [(_H1, _AEVP), (_H2, _H1), (_H3, _H2), (1, _H3)]
    Ws = [[] for _ in range(4)]
    bs = [[] for _ in range(4)]
    for layers in params:
        for j, (W, b) in enumerate(layers):
            ho, hi = dims[j]
            Ws[j].append(jnp.zeros((ho, hi), jnp.float32)
                         .at[:W.shape[0], :W.shape[1]].set(W))
            bs[j].append(jnp.zeros((ho,), jnp.float32).at[:b.shape[0]].set(b))
    for j, (ho, hi) in enumerate(dims):
        Ws[j].append(jnp.zeros((ho, hi), jnp.float32))
        bs[j].append(jnp.zeros((ho,), jnp.float32))
    W1s, W2s, W3s, W4s = (jnp.stack(Ws[j]).astype(jnp.bfloat16)
                          for j in range(4))
    b1s = jnp.stack(bs[0]).reshape(8, 1, _H1)
    b2s = jnp.stack(bs[1]).reshape(8, 1, _H2)
    b3s = jnp.stack(bs[2]).reshape(8, 1, _H3)
    b4s = jnp.stack(bs[3]).reshape(8)
    return W1s, b1s, W2s, b2s, W3s, b3s, W4s, b4s


def kernel(species, aev, params):
    species = species.astype(jnp.int32)
    W1s, b1s, W2s, b2s, W3s, b3s, W4s, b4s = _stack_params(params)

    species2d = jnp.concatenate(
        [species, jnp.full((_NPAD - _N,), 7, jnp.int32)]).reshape(_ROWS, _COLS)

    pos2d, be2d = pl.pallas_call(
        _route_body,
        out_shape=[
            jax.ShapeDtypeStruct((_ROWS, _COLS), jnp.int32),
            jax.ShapeDtypeStruct((1, 512), jnp.int32),
        ],
    )(species2d)

    pos_flat = pos2d.reshape(_NPAD)
    pos_sc = pos_flat[:_N].reshape(_N // _SW, _SW)
    idx_pad = jnp.concatenate(
        [pos_flat, jnp.zeros((_C - _NPAD,), jnp.int32)])
    be_flat = be2d.reshape(512)

    aev_t = aev.T
    aev_p0 = _pad_half(aev_t, 0)
    xs0 = _sc_scatter_rows(aev_p0, pos_sc)
    aev_p1 = _pad_half(aev_t, 1)
    xs1 = _sc_scatter_rows(aev_p1, pos_sc)
    W1sa = W1s[:, :, :_HW]
    W1sb = W1s[:, :, _HW:]

    grid_spec = pltpu.PrefetchScalarGridSpec(
        num_scalar_prefetch=2,
        grid=(_NB,),
        in_specs=[
            pl.BlockSpec((_BM, _HW), lambda k, be, b4: (k, 0)),
            pl.BlockSpec((_BM, _HW), lambda k, be, b4: (k, 0)),
            pl.BlockSpec((1, _H1, _HW), lambda k, be, b4: (be[k], 0, 0)),
            pl.BlockSpec((1, _H1, _HW), lambda k, be, b4: (be[k], 0, 0)),
            pl.BlockSpec((1, 1, _H1), lambda k, be, b4: (be[k], 0, 0)),
            pl.BlockSpec((1, _H2, _H1), lambda k, be, b4: (be[k], 0, 0)),
            pl.BlockSpec((1, 1, _H2), lambda k, be, b4: (be[k], 0, 0)),
            pl.BlockSpec((1, _H3, _H2), lambda k, be, b4: (be[k], 0, 0)),
            pl.BlockSpec((1, 1, _H3), lambda k, be, b4: (be[k], 0, 0)),
            pl.BlockSpec((1, 1, _H3), lambda k, be, b4: (be[k], 0, 0)),
        ],
        out_specs=pl.BlockSpec((1, 1, _BM), lambda k, be, b4: (k, 0, 0)),
    )
    y = pl.pallas_call(
        _mlp_body,
        grid_spec=grid_spec,
        out_shape=jax.ShapeDtypeStruct((_NB, 1, _BM), jnp.float32),
    )(be_flat, b4s, xs0, xs1, W1sa, W1sb, b1s, W2s, b2s, W3s, b3s, W4s)

    out_full = _sc_gather_out(y.reshape(_C), idx_pad)
    return out_full[:_N]


# speedup vs baseline: 1.0580x; 1.0580x over previous
import functools

import jax
import jax.numpy as jnp
from jax.experimental import pallas as pl
from jax.experimental.pallas import tpu as pltpu
from jax.experimental.pallas import tpu_sc as plsc

_AEV = 1008
_AEVP = 1024
_N = 100000
_BM = 512
_NB = 204
_C = _BM * _NB
_ROWS = 8
_COLS = 12512
_NPAD = _ROWS * _COLS
_H1, _H2, _H3 = 256, 192, 160
_SW = 40
_NSUB = 32
_GW = _C // _NSUB


def _celu(x):
    return jnp.where(x > 0, x, 0.1 * (jnp.exp(x * 10.0) - 1.0))


def _cumsum(x, axis):
    n = x.shape[axis]
    k = 1
    while k < n:
        zshape = list(x.shape)
        zshape[axis] = k
        pad = jnp.zeros(zshape, x.dtype)
        if axis == 1:
            x = x + jnp.concatenate([pad, x[:, :n - k]], axis=1)
        else:
            x = x + jnp.concatenate([pad, x[:n - k, :]], axis=0)
        k *= 2
    return x


_PR = 512


def _pad_body(xt_ref, o_ref):
    o_ref[:, :_AEV] = xt_ref[...].T
    o_ref[:, _AEV:] = jnp.zeros((_PR, _AEVP - _AEV), jnp.float32)


def _pad_aev(aev_t):
    nsteps = (_N + _PR - 1) // _PR
    return pl.pallas_call(
        _pad_body,
        grid=(nsteps,),
        in_specs=[pl.BlockSpec((_AEV, _PR), lambda i: (0, i))],
        out_specs=pl.BlockSpec((_PR, _AEVP), lambda i: (i, 0)),
        out_shape=jax.ShapeDtypeStruct((_N, _AEVP), jnp.float32),
    )(aev_t)


def _route_body(s_ref, pos_ref, be_ref):
    s = s_ref[...]
    pos = jnp.zeros(s.shape, jnp.int32)
    off = jnp.int32(0)
    offs = []
    for b in range(8):
        ohb = s == b
        oh = ohb.astype(jnp.float32)
        inc = _cumsum(oh, axis=1)
        row_tot = inc[:, _COLS - 1:_COLS]
        ro = _cumsum(row_tot, axis=0) - row_tot
        rank_excl = inc + ro - oh
        cnt = (ro[7, 0] + row_tot[7, 0]).astype(jnp.int32)
        pos = jnp.where(ohb, off + rank_excl.astype(jnp.int32), pos)
        offs.append(off)
        off = off + ((cnt + _BM - 1) // _BM) * _BM
    pos_ref[...] = pos
    kidx = jax.lax.broadcasted_iota(jnp.int32, (1, 512), 1) * _BM
    be = jnp.zeros((1, 512), jnp.int32)
    for b in range(1, 8):
        be = be + (kidx >= offs[b]).astype(jnp.int32)
    be_ref[...] = be


def _mlp_body(be_ref, b4_ref, x_ref, w1_ref, b1_ref, w2_ref, b2_ref,
              w3_ref, b3_ref, w4_ref, o_ref):
    cdims = (((1,), (1,)), ((), ()))
    x = x_ref[...].astype(jnp.bfloat16)
    h = jax.lax.dot_general(x, w1_ref[0], cdims,
                            preferred_element_type=jnp.float32)
    h = _celu(h + b1_ref[0]).astype(jnp.bfloat16)
    h = jax.lax.dot_general(h, w2_ref[0], cdims,
                            preferred_element_type=jnp.float32)
    h = _celu(h + b2_ref[0]).astype(jnp.bfloat16)
    h = jax.lax.dot_general(h, w3_ref[0], cdims,
                            preferred_element_type=jnp.float32)
    h = _celu(h + b3_ref[0]).astype(jnp.bfloat16)
    y = jax.lax.dot_general(w4_ref[0], h, cdims,
                            preferred_element_type=jnp.float32)
    e = be_ref[pl.program_id(0)]
    o_ref[...] = (y + b4_ref[e]).reshape(1, 1, _BM)


def _sc_scatter_rows(aev, pos_row):
    mesh = plsc.VectorSubcoreMesh(core_axis_name="c", subcore_axis_name="s")

    @functools.partial(
        pl.kernel,
        out_type=jax.ShapeDtypeStruct((_C, _AEVP), jnp.float32),
        mesh=mesh,
        compiler_params=pltpu.CompilerParams(use_tc_tiling_on_sc=True))
    def scat(x_hbm, i_hbm, o_hbm):
        def body(x_vmem, i_vmem):
            pltpu.sync_copy(x_vmem, o_hbm.at[i_vmem.at[0]])

        pltpu.emit_pipeline(
            body,
            grid=(_N // _SW,),
            in_specs=[
                pl.BlockSpec((_SW, _AEVP), lambda i: (i, 0)),
                pl.BlockSpec((1, _SW), lambda i: (i, 0)),
            ],
            out_specs=[],
            core_axis_name=("c", "s"),
            dimension_semantics=(pltpu.PARALLEL,),
        )(x_hbm, i_hbm)

    return scat(aev, pos_row)


def _sc_gather_out(y_flat, idx_pad):
    mesh = plsc.VectorSubcoreMesh(core_axis_name="c", subcore_axis_name="s")

    @functools.partial(
        pl.kernel,
        out_type=jax.ShapeDtypeStruct((_C,), jnp.float32),
        mesh=mesh,
        compiler_params=pltpu.CompilerParams(needs_layout_passes=False),
        scratch_types=[
            pltpu.VMEM((_C,), jnp.float32),
            pltpu.VMEM((_GW,), jnp.int32),
            pltpu.VMEM((_GW,), jnp.float32),
            pltpu.SemaphoreType.DMA,
        ])
    def gat(y_hbm, i_hbm, o_hbm, y_vmem, idx_vmem, out_vmem, sem):
        c = jax.lax.axis_index("c")
        s = jax.lax.axis_index("s")
        u = c * 16 + s
        pltpu.async_copy(y_hbm, y_vmem, sem).wait()
        pltpu.async_copy(i_hbm.at[pl.ds(u * _GW, _GW)], idx_vmem, sem).wait()

        @pl.loop(0, _GW, step=16)
        def _(j):
            idx = idx_vmem[pl.ds(j, 16)]
            out_vmem[pl.ds(j, 16)] = plsc.load_gather(y_vmem, [idx])

        pltpu.async_copy(out_vmem, o_hbm.at[pl.ds(u * _GW, _GW)], sem).wait()

    return gat(y_flat, idx_pad)


def _stack_params(params):
    dims = [(_H1, _AEVP), (_H2, _H1), (_H3, _H2), (1, _H3)]
    Ws = [[] for _ in range(4)]
    bs = [[] for _ in range(4)]
    for layers in params:
        for j, (W, b) in enumerate(layers):
            ho, hi = dims[j]
            Ws[j].append(jnp.zeros((ho, hi), jnp.float32)
                         .at[:W.shape[0], :W.shape[1]].set(W))
            bs[j].append(jnp.zeros((ho,), jnp.float32).at[:b.shape[0]].set(b))
    for j, (ho, hi) in enumerate(dims):
        Ws[j].append(jnp.zeros((ho, hi), jnp.float32))
        bs[j].append(jnp.zeros((ho,), jnp.float32))
    W1s, W2s, W3s, W4s = (jnp.stack(Ws[j]).astype(jnp.bfloat16)
                          for j in range(4))
    b1s = jnp.stack(bs[0]).reshape(8, 1, _H1)
    b2s = jnp.stack(bs[1]).reshape(8, 1, _H2)
    b3s = jnp.stack(bs[2]).reshape(8, 1, _H3)
    b4s = jnp.stack(bs[3]).reshape(8)
    return W1s, b1s, W2s, b2s, W3s, b3s, W4s, b4s


def kernel(species, aev, params):
    species = species.astype(jnp.int32)
    W1s, b1s, W2s, b2s, W3s, b3s, W4s, b4s = _stack_params(params)

    species2d = jnp.concatenate(
        [species, jnp.full((_NPAD - _N,), 7, jnp.int32)]).reshape(_ROWS, _COLS)

    pos2d, be2d = pl.pallas_call(
        _route_body,
        out_shape=[
            jax.ShapeDtypeStruct((_ROWS, _COLS), jnp.int32),
            jax.ShapeDtypeStruct((1, 512), jnp.int32),
        ],
    )(species2d)

    pos_flat = pos2d.reshape(_NPAD)
    pos_sc = pos_flat[:_N].reshape(_N // _SW, _SW)
    idx_pad = jnp.concatenate(
        [pos_flat, jnp.zeros((_C - _NPAD,), jnp.int32)])
    be_flat = be2d.reshape(512)

    aev_p = _pad_aev(aev.T)
    xs = _sc_scatter_rows(aev_p, pos_sc)

    grid_spec = pltpu.PrefetchScalarGridSpec(
        num_scalar_prefetch=2,
        grid=(_NB,),
        in_specs=[
            pl.BlockSpec((_BM, _AEVP), lambda k, be, b4: (k, 0)),
            pl.BlockSpec((1, _H1, _AEVP), lambda k, be, b4: (be[k], 0, 0)),
            pl.BlockSpec((1, 1, _H1), lambda k, be, b4: (be[k], 0, 0)),
            pl.BlockSpec((1, _H2, _H1), lambda k, be, b4: (be[k], 0, 0)),
            pl.BlockSpec((1, 1, _H2), lambda k, be, b4: (be[k], 0, 0)),
            pl.BlockSpec((1, _H3, _H2), lambda k, be, b4: (be[k], 0, 0)),
            pl.BlockSpec((1, 1, _H3), lambda k, be, b4: (be[k], 0, 0)),
            pl.BlockSpec((1, 1, _H3), lambda k, be, b4: (be[k], 0, 0)),
        ],
        out_specs=pl.BlockSpec((1, 1, _BM), lambda k, be, b4: (k, 0, 0)),
    )
    y = pl.pallas_call(
        _mlp_body,
        grid_spec=grid_spec,
        out_shape=jax.ShapeDtypeStruct((_NB, 1, _BM), jnp.float32),
    )(be_flat, b4s, xs, W1s, b1s, W2s, b2s, W3s, b3s, W4s)

    out_full = _sc_gather_out(y.reshape(_C), idx_pad)
    return out_full[:_N]
